# baseline (speedup 1.0000x reference)
import jax
import jax.numpy as jnp
from jax import lax
from jax.experimental import pallas as pl
from jax.experimental.pallas import tpu as pltpu

MAX_F, MAX_T = 64, 512
N_EMBD = 512
N_EMBD_F = 256
N_EMBD_T = 256

_CHUNK = 8
_NBUF = 3


def _pos_kernel(f_ref, t_ref, x_hbm, o_hbm, in_buf, out_buf, in_sem, out_sem):
    n_rows = x_hbm.shape[0]
    n_chunks = n_rows // _CHUNK
    tb = t_ref[...]

    def start_in(i, slot):
        pltpu.make_async_copy(
            x_hbm.at[pl.ds(i * _CHUNK, _CHUNK)], in_buf.at[slot], in_sem.at[slot]
        ).start()

    for s in range(_NBUF):
        start_in(s, s)

    def body(i, _):
        slot = lax.rem(i, _NBUF)
        pltpu.make_async_copy(
            x_hbm.at[pl.ds(i * _CHUNK, _CHUNK)], in_buf.at[slot], in_sem.at[slot]
        ).wait()

        @pl.when(i >= _NBUF)
        def _():
            pltpu.make_async_copy(
                out_buf.at[slot],
                o_hbm.at[pl.ds((i - _NBUF) * _CHUNK, _CHUNK)],
                out_sem.at[slot],
            ).wait()

        f_start = lax.rem(i, MAX_F // _CHUNK) * _CHUNK
        fb = f_ref[pl.ds(f_start, _CHUNK), :]
        out_buf[slot, :, :, :N_EMBD_F] = in_buf[slot, :, :, :N_EMBD_F] + fb[:, None, :]
        out_buf[slot, :, :, N_EMBD_F:] = in_buf[slot, :, :, N_EMBD_F:] + tb[None, :, :]

        pltpu.make_async_copy(
            out_buf.at[slot], o_hbm.at[pl.ds(i * _CHUNK, _CHUNK)], out_sem.at[slot]
        ).start()

        @pl.when(i + _NBUF < n_chunks)
        def _():
            start_in(i + _NBUF, slot)

        return 0

    lax.fori_loop(0, n_chunks, body, 0)

    for k in range(_NBUF):
        i = n_chunks - _NBUF + k
        pltpu.make_async_copy(
            out_buf.at[i % _NBUF],
            o_hbm.at[pl.ds(i * _CHUNK, _CHUNK)],
            out_sem.at[i % _NBUF],
        ).wait()


def kernel(x, freq_embed, time_embed):
    B, F, T, D = x.shape
    xf = x.reshape(B * F, T, D)
    out = pl.pallas_call(
        _pos_kernel,
        in_specs=[
            pl.BlockSpec(memory_space=pltpu.MemorySpace.VMEM),
            pl.BlockSpec(memory_space=pltpu.MemorySpace.VMEM),
            pl.BlockSpec(memory_space=pltpu.MemorySpace.HBM),
        ],
        out_specs=pl.BlockSpec(memory_space=pltpu.MemorySpace.HBM),
        out_shape=jax.ShapeDtypeStruct(xf.shape, x.dtype),
        scratch_shapes=[
            pltpu.VMEM((_NBUF, _CHUNK, T, D), x.dtype),
            pltpu.VMEM((_NBUF, _CHUNK, T, D), x.dtype),
            pltpu.SemaphoreType.DMA((_NBUF,)),
            pltpu.SemaphoreType.DMA((_NBUF,)),
        ],
    )(freq_embed, time_embed, xf)
    return out.reshape(B, F, T, D)
